# Initial kernel scaffold; baseline (speedup 1.0000x reference)
import jax
import jax.numpy as jnp
from jax.experimental import pallas as pl


def kernel(user, traj, geo, long_traj, traj_graph_x, geo_graph_x, user_table, loc_table, geo_table):
    raise NotImplementedError("write your pallas kernel here")


# speedup vs baseline: 1.0386x; 1.0386x over previous
import functools

import jax
import jax.numpy as jnp
from jax import lax
from jax.experimental import pallas as pl
from jax.experimental.pallas import tpu as pltpu
from jax.experimental.pallas import tpu_sc as plsc

HIDDEN = 64
CHUNK = 128

NC = 2
NS = 16
NW = NC * NS


def _pad_chunks(idx):
    flat = idx.reshape(-1).astype(jnp.int32)
    n = flat.shape[0]
    n_pad = -n % CHUNK
    if n_pad:
        flat = jnp.concatenate([flat, jnp.zeros((n_pad,), jnp.int32)])
    return flat.reshape(-1, CHUNK), n


def _make_sc_kernel(task_specs):
    mesh = plsc.VectorSubcoreMesh(core_axis_name="c", subcore_axis_name="s")

    out_type = tuple(
        jax.ShapeDtypeStruct((nc * CHUNK, HIDDEN), jnp.float32)
        for _, nc in task_specs
    )

    def body(user_table, loc_table, geo_table, *rest):
        n_tasks = len(task_specs)
        idx_refs = rest[:n_tasks]
        out_refs = rest[n_tasks:2 * n_tasks]
        idx_v, rows_v, sem = rest[2 * n_tasks:]
        tables = (user_table, loc_table, geo_table)

        wid = lax.axis_index("s") * NC + lax.axis_index("c")

        for (tslot, n_chunks), idx_hbm, out_hbm in zip(
                task_specs, idx_refs, out_refs):
            table = tables[tslot]
            n_iter = (n_chunks + NW - 1) // NW

            def step(i, _, idx_hbm=idx_hbm, out_hbm=out_hbm, table=table,
                     n_chunks=n_chunks):
                c = wid + i * NW

                @pl.when(c < n_chunks)
                def _():
                    pltpu.sync_copy(idx_hbm.at[c], idx_v)
                    pltpu.async_copy(table.at[idx_v], rows_v, sem).wait()
                    pltpu.sync_copy(rows_v, out_hbm.at[pl.ds(c * CHUNK, CHUNK)])

                return ()

            lax.fori_loop(0, n_iter, step, ())

    return pl.kernel(
        body,
        out_type=out_type,
        mesh=mesh,
        compiler_params=pltpu.CompilerParams(use_tc_tiling_on_sc=False),
        scratch_types=[
            pltpu.VMEM((CHUNK,), jnp.int32),
            pltpu.VMEM((CHUNK, HIDDEN), jnp.float32),
            pltpu.SemaphoreType.DMA,
        ],
    )


def kernel(user, traj, geo, long_traj, traj_graph_x, geo_graph_x,
           user_table, loc_table, geo_table):
    idx_arrays = []
    true_lens = []
    for idx in (user, traj, geo, long_traj, traj_graph_x, geo_graph_x):
        chunks, n = _pad_chunks(idx)
        idx_arrays.append(chunks)
        true_lens.append(n)

    tslots = (0, 1, 2, 1, 1, 2)
    task_specs = tuple(
        (t, a.shape[0]) for t, a in zip(tslots, idx_arrays)
    )

    sc = _make_sc_kernel(task_specs)
    outs = sc(user_table, loc_table, geo_table, *idx_arrays)

    shapes = (
        user.shape + (HIDDEN,),
        traj.shape + (HIDDEN,),
        geo.shape + (HIDDEN,),
        long_traj.shape + (HIDDEN,),
        traj_graph_x.shape + (HIDDEN,),
        geo_graph_x.shape + (HIDDEN,),
    )
    return tuple(
        o[:n].reshape(shape)
        for o, n, shape in zip(outs, true_lens, shapes)
    )


# speedup vs baseline: 1.0956x; 1.0548x over previous
import functools

import jax
import jax.numpy as jnp
from jax import lax
from jax.experimental import pallas as pl
from jax.experimental.pallas import tpu as pltpu
from jax.experimental.pallas import tpu_sc as plsc

HIDDEN = 64
CHUNK = 128
KMAX = 6

NC = 2
NS = 16
NW = NC * NS


def _plan(n_rows):
    n_chunks = -(-n_rows // CHUNK)
    cpw = -(-n_chunks // NW)
    best = None
    for k in range(1, KMAX + 1):
        gw = -(-cpw // k)
        gw += gw % 2
        span = gw * k
        cost = span * 1.2 + gw * 2.0
        if best is None or cost < best[0]:
            best = (cost, k, gw, span)
    _, k, gw, span = best
    return k, gw, span


def _make_sc_kernel(task_specs):
    mesh = plsc.VectorSubcoreMesh(core_axis_name="c", subcore_axis_name="s")

    out_type = tuple(
        jax.ShapeDtypeStruct((npad, HIDDEN), jnp.float32)
        for _, npad, _, _ in task_specs
    )

    def body(user_table, loc_table, geo_table, *rest):
        n_tasks = len(task_specs)
        idx_refs = rest[:n_tasks]
        out_refs = rest[n_tasks:2 * n_tasks]
        idx_v, rows0, rows1, sem_g, sem_w0, sem_w1 = rest[2 * n_tasks:]
        tables = (user_table, loc_table, geo_table)

        wid = lax.axis_index("s") * NC + lax.axis_index("c")

        for (tslot, npad, K, gw), idx_hbm, out_hbm in zip(
                task_specs, idx_refs, out_refs):
            table = tables[tslot]
            span = gw * K
            grows = K * CHUNK
            c0 = wid * span
            hb = gw // 2

            def wr_desc(rows_v, sem, goff):
                return pltpu.make_async_copy(
                    rows_v.at[pl.ds(0, grows)],
                    out_hbm.at[pl.ds(goff * CHUNK, grows)],
                    sem)

            def run_group(h, rows_v, sem_w, parity, idx_base):
                g = h * 2 + parity
                goff = c0 + g * K

                @pl.when(h >= 1)
                def _():
                    wr_desc(rows_v, sem_w, goff - 2 * K).wait()

                for b in range(K):
                    pltpu.make_async_copy(
                        table.at[idx_v.at[idx_base + b]],
                        rows_v.at[pl.ds(b * CHUNK, CHUNK)],
                        sem_g).start()
                for b in range(K):
                    pltpu.make_async_copy(
                        table.at[idx_v.at[idx_base + b]],
                        rows_v.at[pl.ds(b * CHUNK, CHUNK)],
                        sem_g).wait()
                wr_desc(rows_v, sem_w, goff).start()

            def step(h, _):
                pltpu.sync_copy(
                    idx_hbm.at[pl.ds(c0 + h * 2 * K, 2 * K)],
                    idx_v.at[pl.ds(0, 2 * K)])
                run_group(h, rows0, sem_w0, 0, 0)
                run_group(h, rows1, sem_w1, 1, K)
                return ()

            lax.fori_loop(0, hb, step, ())

            wr_desc(rows0, sem_w0, c0 + (gw - 2) * K).wait()
            wr_desc(rows1, sem_w1, c0 + (gw - 1) * K).wait()

    return pl.kernel(
        body,
        out_type=out_type,
        mesh=mesh,
        compiler_params=pltpu.CompilerParams(use_tc_tiling_on_sc=False),
        scratch_types=[
            pltpu.VMEM((2 * KMAX, CHUNK), jnp.int32),
            pltpu.VMEM((KMAX * CHUNK, HIDDEN), jnp.float32),
            pltpu.VMEM((KMAX * CHUNK, HIDDEN), jnp.float32),
            pltpu.SemaphoreType.DMA,
            pltpu.SemaphoreType.DMA,
            pltpu.SemaphoreType.DMA,
        ],
    )


def kernel(user, traj, geo, long_traj, traj_graph_x, geo_graph_x,
           user_table, loc_table, geo_table):
    srcs = (user, traj, geo, long_traj, traj_graph_x, geo_graph_x)
    tslots = (0, 1, 2, 1, 1, 2)
    tsizes = (user_table.shape[0], loc_table.shape[0], geo_table.shape[0])

    idx_arrays = []
    true_lens = []
    task_specs = []
    for idx, tslot in zip(srcs, tslots):
        flat = idx.reshape(-1).astype(jnp.int32)
        n = flat.shape[0]
        K, gw, span = _plan(n)
        npad = NW * span * CHUNK
        if npad > n:
            pad = jnp.arange(npad - n, dtype=jnp.int32) % tsizes[tslot]
            flat = jnp.concatenate([flat, pad])
        idx_arrays.append(flat.reshape(-1, CHUNK))
        true_lens.append(n)
        task_specs.append((tslot, npad, K, gw))

    sc = _make_sc_kernel(tuple(task_specs))
    outs = sc(user_table, loc_table, geo_table, *idx_arrays)

    return tuple(
        o[:n].reshape(src.shape + (HIDDEN,))
        for o, n, src in zip(outs, true_lens, srcs)
    )


# speedup vs baseline: 1.1856x; 1.0822x over previous
import functools

import jax
import jax.numpy as jnp
from jax import lax
from jax.experimental import pallas as pl
from jax.experimental.pallas import tpu as pltpu
from jax.experimental.pallas import tpu_sc as plsc

HIDDEN = 64
CHUNK = 128
KMAX = 6

NC = 2
NS = 16
NW = NC * NS


def _plan(n_rows):
    n_chunks = -(-n_rows // CHUNK)
    cpw = -(-n_chunks // NW)
    best = None
    for k in range(1, KMAX + 1):
        gw = -(-cpw // k)
        gw += gw % 2
        span = gw * k
        cost = span * 1.2 + gw * 2.0
        if best is None or cost < best[0]:
            best = (cost, k, gw, span)
    _, k, gw, span = best
    return k, gw, span


def _make_sc_kernel(task_specs):
    mesh = plsc.VectorSubcoreMesh(core_axis_name="c", subcore_axis_name="s")

    out_type = tuple(
        jax.ShapeDtypeStruct((npad, HIDDEN), jnp.float32)
        for _, npad, _, _ in task_specs
    )

    def body(user_table, loc_table, geo_table, *rest):
        n_tasks = len(task_specs)
        idx_refs = rest[:n_tasks]
        out_refs = rest[n_tasks:2 * n_tasks]
        idx_v, rows0, rows1, sem_g, sem_w0, sem_w1 = rest[2 * n_tasks:]
        tables = (user_table, loc_table, geo_table)

        wid = lax.axis_index("s") * NC + lax.axis_index("c")

        for (tslot, npad, K, gw), idx_hbm, out_hbm in zip(
                task_specs, idx_refs, out_refs):
            table = tables[tslot]
            span = gw * K
            grows = K * CHUNK
            c0 = wid * span
            hb = gw // 2

            def wr_desc(rows_v, sem, goff):
                return pltpu.make_async_copy(
                    rows_v.at[pl.ds(0, grows)],
                    out_hbm.at[pl.ds(goff * CHUNK, grows)],
                    sem)

            def run_group(h, rows_v, sem_w, parity, idx_base):
                g = h * 2 + parity
                goff = c0 + g * K

                @pl.when(h >= 1)
                def _():
                    wr_desc(rows_v, sem_w, goff - 2 * K).wait()

                for b in range(K):
                    pltpu.make_async_copy(
                        table.at[idx_v.at[idx_base + b]],
                        rows_v.at[pl.ds(b * CHUNK, CHUNK)],
                        sem_g).start()
                for b in range(K):
                    pltpu.make_async_copy(
                        table.at[idx_v.at[idx_base + b]],
                        rows_v.at[pl.ds(b * CHUNK, CHUNK)],
                        sem_g).wait()
                wr_desc(rows_v, sem_w, goff).start()

            def step(h, _):
                pltpu.sync_copy(
                    idx_hbm.at[pl.ds(c0 + h * 2 * K, 2 * K)],
                    idx_v.at[pl.ds(0, 2 * K)])
                run_group(h, rows0, sem_w0, 0, 0)
                run_group(h, rows1, sem_w1, 1, K)
                return ()

            lax.fori_loop(0, hb, step, ())

            wr_desc(rows0, sem_w0, c0 + (gw - 2) * K).wait()
            wr_desc(rows1, sem_w1, c0 + (gw - 1) * K).wait()

    return pl.kernel(
        body,
        out_type=out_type,
        mesh=mesh,
        compiler_params=pltpu.CompilerParams(use_tc_tiling_on_sc=False),
        scratch_types=[
            pltpu.VMEM((2 * KMAX, CHUNK), jnp.int32),
            pltpu.VMEM((KMAX * CHUNK, HIDDEN), jnp.float32),
            pltpu.VMEM((KMAX * CHUNK, HIDDEN), jnp.float32),
            pltpu.SemaphoreType.DMA,
            pltpu.SemaphoreType.DMA,
            pltpu.SemaphoreType.DMA,
        ],
    )


def kernel(user, traj, geo, long_traj, traj_graph_x, geo_graph_x,
           user_table, loc_table, geo_table):
    srcs = (user, traj, geo, long_traj, traj_graph_x, geo_graph_x)
    tslots = (0, 1, 2, 1, 1, 2)
    tsizes = (user_table.shape[0], loc_table.shape[0], geo_table.shape[0])

    idx_arrays = []
    true_lens = []
    task_specs = []
    for idx, tslot in zip(srcs, tslots):
        flat = idx.reshape(-1).astype(jnp.int32)
        n = flat.shape[0]
        K, gw, span = _plan(n)
        npad = NW * span * CHUNK
        if npad > n:
            pad = jnp.arange(npad - n, dtype=jnp.int32) % tsizes[tslot]
            flat = jnp.concatenate([flat, pad])
        idx_arrays.append(flat.reshape(-1, CHUNK))
        true_lens.append(n)
        task_specs.append((tslot, npad, K, gw))

    def to_sc(t):
        v = t.shape[0]
        return lax.optimization_barrier(
            t.reshape(v // 2, 2 * HIDDEN)).reshape(v, HIDDEN)

    sc = _make_sc_kernel(tuple(task_specs))
    outs = sc(to_sc(user_table), to_sc(loc_table), to_sc(geo_table),
              *idx_arrays)

    def from_sc(o, n, shape):
        npad = o.shape[0]
        o = lax.optimization_barrier(o.reshape(npad // 2, 2 * HIDDEN))
        return o.reshape(npad, HIDDEN)[:n].reshape(shape)

    return tuple(
        from_sc(o, n, src.shape + (HIDDEN,))
        for o, n, src in zip(outs, true_lens, srcs)
    )


# speedup vs baseline: 1.2286x; 1.0362x over previous
import functools

import jax
import jax.numpy as jnp
from jax import lax
from jax.experimental import pallas as pl
from jax.experimental.pallas import tpu as pltpu
from jax.experimental.pallas import tpu_sc as plsc

HIDDEN = 64
CHUNK = 128
KMAX = 6

NC = 2
NS = 16
NW = NC * NS


def _plan(n_rows):
    n_chunks = -(-n_rows // CHUNK)
    cpw = -(-n_chunks // NW)
    best = None
    for k in range(1, KMAX + 1):
        gw = -(-cpw // k)
        gw += gw % 2
        span = gw * k
        cost = span * 1.2 + gw * 2.0
        if best is None or cost < best[0]:
            best = (cost, k, gw)
    _, k, gw = best
    return k, gw


@functools.lru_cache(maxsize=None)
def _make_sc_gather(n_pad_rows, K, gw):
    mesh = plsc.VectorSubcoreMesh(core_axis_name="c", subcore_axis_name="s")

    def body(table, idx_hbm, out_hbm, idx_v, rows0, rows1,
             sem_g, sem_w0, sem_w1):
        wid = lax.axis_index("s") * NC + lax.axis_index("c")
        span = gw * K
        grows = K * CHUNK
        c0 = wid * span
        hb = gw // 2

        def wr_desc(rows_v, sem, goff):
            return pltpu.make_async_copy(
                rows_v.at[pl.ds(0, grows)],
                out_hbm.at[pl.ds(goff * CHUNK, grows)],
                sem)

        def run_group(h, rows_v, sem_w, parity, idx_base):
            g = h * 2 + parity
            goff = c0 + g * K

            @pl.when(h >= 1)
            def _():
                wr_desc(rows_v, sem_w, goff - 2 * K).wait()

            for b in range(K):
                pltpu.make_async_copy(
                    table.at[idx_v.at[idx_base + b]],
                    rows_v.at[pl.ds(b * CHUNK, CHUNK)],
                    sem_g).start()
            for b in range(K):
                pltpu.make_async_copy(
                    table.at[idx_v.at[idx_base + b]],
                    rows_v.at[pl.ds(b * CHUNK, CHUNK)],
                    sem_g).wait()
            wr_desc(rows_v, sem_w, goff).start()

        def step(h, _):
            pltpu.sync_copy(
                idx_hbm.at[pl.ds(c0 + h * 2 * K, 2 * K)],
                idx_v.at[pl.ds(0, 2 * K)])
            run_group(h, rows0, sem_w0, 0, 0)
            run_group(h, rows1, sem_w1, 1, K)
            return ()

        lax.fori_loop(0, hb, step, ())

        wr_desc(rows0, sem_w0, c0 + (gw - 2) * K).wait()
        wr_desc(rows1, sem_w1, c0 + (gw - 1) * K).wait()

    return pl.kernel(
        body,
        out_type=jax.ShapeDtypeStruct((n_pad_rows, HIDDEN), jnp.float32),
        mesh=mesh,
        compiler_params=pltpu.CompilerParams(use_tc_tiling_on_sc=False),
        scratch_types=[
            pltpu.VMEM((2 * KMAX, CHUNK), jnp.int32),
            pltpu.VMEM((KMAX * CHUNK, HIDDEN), jnp.float32),
            pltpu.VMEM((KMAX * CHUNK, HIDDEN), jnp.float32),
            pltpu.SemaphoreType.DMA,
            pltpu.SemaphoreType.DMA,
            pltpu.SemaphoreType.DMA,
        ],
    )


def kernel(user, traj, geo, long_traj, traj_graph_x, geo_graph_x,
           user_table, loc_table, geo_table):
    tables = (user_table, loc_table, geo_table)
    srcs = (user, traj, geo, long_traj, traj_graph_x, geo_graph_x)
    tslots = (0, 1, 2, 1, 1, 2)

    order = (0, 2, 5, 1, 3, 4)

    outs = [None] * len(srcs)
    for t in order:
        src, tslot = srcs[t], tslots[t]
        table = tables[tslot]
        flat = src.reshape(-1).astype(jnp.int32)
        n = flat.shape[0]
        K, gw = _plan(n)
        npad = NW * gw * K * CHUNK
        if npad > n:
            pad = jnp.arange(npad - n, dtype=jnp.int32) % table.shape[0]
            flat = jnp.concatenate([flat, pad])
        idx2d = flat.reshape(-1, CHUNK)

        o = _make_sc_gather(npad, K, gw)(table, idx2d)
        o = lax.optimization_barrier(o.reshape(npad // 2, 2 * HIDDEN))
        outs[t] = o.reshape(npad, HIDDEN)[:n].reshape(src.shape + (HIDDEN,))

    return tuple(outs)
